# Initial kernel scaffold; baseline (speedup 1.0000x reference)
import jax
import jax.numpy as jnp
from jax.experimental import pallas as pl


def kernel(x_ligand, edge_index_ligand, x_protein, edge_index_protein, Wl1, Wl2, Wp1, Wp2, Wqkv, Wo, Wfc, Wout):
    raise NotImplementedError("write your pallas kernel here")


# speedup vs baseline: 4.5899x; 4.5899x over previous
import functools

import jax
import jax.numpy as jnp
from jax import lax
from jax.experimental import pallas as pl
from jax.experimental.pallas import tpu as pltpu
from jax.experimental.pallas import tpu_sc as plsc

_N = 10000
_D = 128
_SEQ = 512
_HALF = _SEQ // 2
_HEADS = 4
_HEAD_DIM = _D // _HEADS
_HIDDEN = 64

_NC = 2
_NS = 16
_NW = _NC * _NS
_K = 128
_N_ACC = 10112
_N_DUMP = _N_ACC - _N
_RPT = _N_ACC // _NS

_CHUNK_EDGES = _NW * _K


def _round_up(x, m):
    return ((x + m - 1) // m) * m


@functools.lru_cache(maxsize=None)
def _make_agg(e_pad: int):
    epw = e_pad // _NW
    nchunks = epw // _K
    mesh = plsc.VectorSubcoreMesh(core_axis_name="c", subcore_axis_name="s")

    out_type = jax.ShapeDtypeStruct((_NC, _N_ACC, _D), jnp.float32)
    scratch = (
        pltpu.VMEM((_K,), jnp.int32),
        pltpu.VMEM((_K,), jnp.int32),
        pltpu.VMEM((_K, _D), jnp.float32),
        pltpu.VMEM_SHARED((_N_ACC, _D), jnp.float32),
        pltpu.SemaphoreType.DMA,
    )

    @functools.partial(pl.kernel, out_type=out_type, mesh=mesh,
                       scratch_types=scratch)
    def agg(y, srcs, dsts, zrow, out_acc, src_v, dst_v, rows_v, acc_sh, sem):
        c = lax.axis_index("c")
        s = lax.axis_index("s")
        r0 = s * _RPT
        pltpu.sync_copy(zrow.at[pl.ds(r0, _RPT)], acc_sh.at[pl.ds(r0, _RPT)])
        plsc.subcore_barrier()
        base = (c * _NS + s) * epw

        def chunk(i, carry):
            off = pl.multiple_of(base + i * _K, 8)
            pltpu.sync_copy(srcs.at[pl.ds(off, _K)], src_v)
            pltpu.sync_copy(dsts.at[pl.ds(off, _K)], dst_v)
            pltpu.async_copy(y.at[src_v], rows_v, sem).wait()
            pltpu.sync_copy(rows_v, acc_sh.at[dst_v], add=True)
            return carry

        lax.fori_loop(0, nchunks, chunk, 0)
        plsc.subcore_barrier()
        pltpu.sync_copy(acc_sh.at[pl.ds(r0, _RPT)],
                        out_acc.at[c, pl.ds(r0, _RPT)])

    return agg


@functools.lru_cache(maxsize=None)
def _make_deg(e_pad: int):
    epw = e_pad // _NW
    nchunks = epw // _K
    mesh = plsc.VectorSubcoreMesh(core_axis_name="c", subcore_axis_name="s")

    out_type = jax.ShapeDtypeStruct((_NC, _N_ACC, _D), jnp.float32)
    scratch = (
        pltpu.VMEM((_K,), jnp.int32),
        pltpu.VMEM((_K, _D), jnp.float32),
        pltpu.VMEM_SHARED((_N_ACC, _D), jnp.float32),
    )

    @functools.partial(pl.kernel, out_type=out_type, mesh=mesh,
                       scratch_types=scratch)
    def deg(dsts, zrow, ones, out_deg, dst_v, ones_v, deg_sh):
        c = lax.axis_index("c")
        s = lax.axis_index("s")
        r0 = s * _RPT
        pltpu.sync_copy(zrow.at[pl.ds(r0, _RPT)], deg_sh.at[pl.ds(r0, _RPT)])
        pltpu.sync_copy(ones, ones_v)
        plsc.subcore_barrier()
        base = (c * _NS + s) * epw

        def chunk(i, carry):
            off = pl.multiple_of(base + i * _K, 8)
            pltpu.sync_copy(dsts.at[pl.ds(off, _K)], dst_v)
            pltpu.sync_copy(ones_v, deg_sh.at[dst_v], add=True)
            return carry

        lax.fori_loop(0, nchunks, chunk, 0)
        plsc.subcore_barrier()
        pltpu.sync_copy(deg_sh.at[pl.ds(r0, _RPT)],
                        out_deg.at[c, pl.ds(r0, _RPT)])

    return deg


def _pad_edges(edge_index, e_pad):
    pad = e_pad - edge_index.shape[1]
    fill = jnp.arange(pad, dtype=jnp.int32)
    src = jnp.concatenate([edge_index[0], fill % 512])
    dst = jnp.concatenate([edge_index[1], _N + (fill % _N_DUMP)])
    return src, dst


def _matmul(x, w):
    n, d = x.shape
    f = w.shape[1]
    blk = 1000

    def body(x_ref, w_ref, o_ref):
        o_ref[...] = jnp.dot(x_ref[...], w_ref[...],
                             preferred_element_type=jnp.float32)

    return pl.pallas_call(
        body,
        grid=(n // blk,),
        in_specs=[pl.BlockSpec((blk, d), lambda i: (i, 0)),
                  pl.BlockSpec((d, f), lambda i: (0, 0))],
        out_specs=pl.BlockSpec((blk, f), lambda i: (i, 0)),
        out_shape=jax.ShapeDtypeStruct((n, f), jnp.float32),
    )(x, w)


def _mid(acc, deg, w):
    blk = 1000

    def body(a_ref, d_ref, w_ref, o_ref):
        sm = a_ref[0] + a_ref[1]
        dg = d_ref[0, :, 0:1] + d_ref[1, :, 0:1]
        h = jnp.maximum(sm / jnp.maximum(dg, 1.0), 0.0)
        o_ref[...] = jnp.dot(h, w_ref[...], preferred_element_type=jnp.float32)

    return pl.pallas_call(
        body,
        grid=(_N // blk,),
        in_specs=[pl.BlockSpec((2, blk, _D), lambda i: (0, i, 0)),
                  pl.BlockSpec((2, blk, _D), lambda i: (0, i, 0)),
                  pl.BlockSpec((_D, _D), lambda i: (0, 0))],
        out_specs=pl.BlockSpec((blk, _D), lambda i: (i, 0)),
        out_shape=jax.ShapeDtypeStruct((_N, _D), jnp.float32),
    )(acc, deg, w)


def _final(accl, degl, accp, degp, wq, wk, wv, wo, wfc, wout):
    def body(al, dl, ap, dp, wq_r, wk_r, wv_r, wo_r, wfc_r, wout_r, o_ref):
        hl = jnp.maximum((al[0] + al[1])
                         / jnp.maximum(dl[0, :, 0:1] + dl[1, :, 0:1], 1.0), 0.0)
        hp = jnp.maximum((ap[0] + ap[1])
                         / jnp.maximum(dp[0, :, 0:1] + dp[1, :, 0:1], 1.0), 0.0)
        seq = jnp.concatenate([hl, hp], axis=0)
        scale = _HEAD_DIM ** -0.5
        pooled = jnp.zeros((1, _D), dtype=jnp.float32)
        for h in range(_HEADS):
            q = jnp.dot(seq, wq_r[h], preferred_element_type=jnp.float32)
            k = jnp.dot(seq, wk_r[h], preferred_element_type=jnp.float32)
            v = jnp.dot(seq, wv_r[h], preferred_element_type=jnp.float32)
            sc = lax.dot_general(q, k, (((1,), (1,)), ((), ())),
                                 preferred_element_type=jnp.float32) * scale
            sc = sc - jnp.max(sc, axis=-1, keepdims=True)
            e = jnp.exp(sc)
            p = e / jnp.sum(e, axis=-1, keepdims=True)
            oh = jnp.dot(p, v, preferred_element_type=jnp.float32)
            pooled = pooled + jnp.dot(jnp.mean(oh, axis=0, keepdims=True),
                                      wo_r[h], preferred_element_type=jnp.float32)
        hh = jnp.maximum(jnp.dot(pooled, wfc_r[...],
                                 preferred_element_type=jnp.float32), 0.0)
        logit = jnp.dot(hh, wout_r[...], preferred_element_type=jnp.float32)
        o_ref[...] = 1.0 / (1.0 + jnp.exp(-logit))

    no_idx = lambda i: (0, 0, 0)
    return pl.pallas_call(
        body,
        grid=(1,),
        in_specs=[pl.BlockSpec((2, _HALF, _D), no_idx),
                  pl.BlockSpec((2, _HALF, _D), no_idx),
                  pl.BlockSpec((2, _HALF, _D), no_idx),
                  pl.BlockSpec((2, _HALF, _D), no_idx),
                  pl.BlockSpec((_HEADS, _D, _HEAD_DIM), no_idx),
                  pl.BlockSpec((_HEADS, _D, _HEAD_DIM), no_idx),
                  pl.BlockSpec((_HEADS, _D, _HEAD_DIM), no_idx),
                  pl.BlockSpec((_HEADS, _HEAD_DIM, _D), no_idx),
                  pl.BlockSpec((_D, _HIDDEN), lambda i: (0, 0)),
                  pl.BlockSpec((_HIDDEN, 128), lambda i: (0, 0))],
        out_specs=pl.BlockSpec((1, 128), lambda i: (0, 0)),
        out_shape=jax.ShapeDtypeStruct((1, 128), jnp.float32),
    )(accl, degl, accp, degp, wq, wk, wv, wo, wfc, wout)


def kernel(x_ligand, edge_index_ligand, x_protein, edge_index_protein,
           Wl1, Wl2, Wp1, Wp2, Wqkv, Wo, Wfc, Wout):
    e_l = edge_index_ligand.shape[1]
    e_p = edge_index_protein.shape[1]
    epad_l = _round_up(e_l, _CHUNK_EDGES)
    epad_p = _round_up(e_p, _CHUNK_EDGES)

    src_l, dst_l = _pad_edges(edge_index_ligand, epad_l)
    src_p, dst_p = _pad_edges(edge_index_protein, epad_p)

    zrow = jnp.zeros((_N_ACC, _D), jnp.float32)
    ones = jnp.zeros((_K, _D), jnp.float32).at[:, 0].set(1.0)

    deg_l = _make_deg(epad_l)(dst_l, zrow, ones)
    deg_p = _make_deg(epad_p)(dst_p, zrow, ones)

    y_l = _matmul(x_ligand, Wl1)
    y_p = _matmul(x_protein, Wp1)
    acc_l1 = _make_agg(epad_l)(y_l, src_l, dst_l, zrow)
    acc_p1 = _make_agg(epad_p)(y_p, src_p, dst_p, zrow)

    z_l = _mid(acc_l1, deg_l, Wl2)
    z_p = _mid(acc_p1, deg_p, Wp2)

    acc_l2 = _make_agg(epad_l)(z_l, src_l, dst_l, zrow)
    acc_p2 = _make_agg(epad_p)(z_p, src_p, dst_p, zrow)

    wq = Wqkv[:, :_D].reshape(_D, _HEADS, _HEAD_DIM).transpose(1, 0, 2)
    wk = Wqkv[:, _D:2 * _D].reshape(_D, _HEADS, _HEAD_DIM).transpose(1, 0, 2)
    wv = Wqkv[:, 2 * _D:].reshape(_D, _HEADS, _HEAD_DIM).transpose(1, 0, 2)
    wo = Wo.reshape(_HEADS, _HEAD_DIM, _D)
    wout = jnp.pad(Wout, ((0, 0), (0, 127)))

    out = _final(acc_l2, deg_l, acc_p2, deg_p, wq, wk, wv, wo, Wfc, wout)
    return out[0, 0:1]


# speedup vs baseline: 6.5084x; 1.4180x over previous
import functools

import jax
import jax.numpy as jnp
from jax import lax
from jax.experimental import pallas as pl
from jax.experimental.pallas import tpu as pltpu
from jax.experimental.pallas import tpu_sc as plsc

_N = 10000
_D = 128
_SEQ = 512
_HALF = _SEQ // 2
_HEADS = 4
_HEAD_DIM = _D // _HEADS
_HIDDEN = 64

_NC = 2
_NS = 16
_NW = _NC * _NS
_K = 128
_N_ACC = 10112
_N_DUMP = _N_ACC - _N
_RPT = _N_ACC // _NS

_CHUNK_EDGES = _NW * _K


def _round_up(x, m):
    return ((x + m - 1) // m) * m


@functools.lru_cache(maxsize=None)
def _make_agg(e_pad: int):
    epw = e_pad // _NW
    nchunks = epw // _K
    assert nchunks % 2 == 0
    mesh = plsc.VectorSubcoreMesh(core_axis_name="c", subcore_axis_name="s")

    out_type = jax.ShapeDtypeStruct((_NC, _N_ACC, _D), jnp.float32)
    scratch = (
        pltpu.VMEM((2, _K), jnp.int32),
        pltpu.VMEM((2, _K), jnp.int32),
        pltpu.VMEM((2, _K, _D), jnp.float32),
        pltpu.VMEM_SHARED((_N_ACC, _D), jnp.float32),
        pltpu.SemaphoreType.DMA,
        pltpu.SemaphoreType.DMA,
    )

    @functools.partial(pl.kernel, out_type=out_type, mesh=mesh,
                       scratch_types=scratch)
    def agg(y, srcs, dsts, zrow, out_acc, src_v, dst_v, rows_v, acc_sh,
            sem0, sem1):
        c = lax.axis_index("c")
        s = lax.axis_index("s")
        r0 = s * _RPT
        pltpu.sync_copy(zrow.at[pl.ds(r0, _RPT)], acc_sh.at[pl.ds(r0, _RPT)])
        plsc.subcore_barrier()
        base = (c * _NS + s) * epw
        sems = (sem0, sem1)

        def start_gather(i, b):
            off = pl.multiple_of(base + i * _K, 8)
            pltpu.sync_copy(srcs.at[pl.ds(off, _K)], src_v.at[b])
            pltpu.sync_copy(dsts.at[pl.ds(off, _K)], dst_v.at[b])
            return pltpu.async_copy(y.at[src_v.at[b]], rows_v.at[b], sems[b])

        def drain(i, b):
            pltpu.make_async_copy(y.at[src_v.at[b]], rows_v.at[b],
                                  sems[b]).wait()
            pltpu.sync_copy(rows_v.at[b], acc_sh.at[dst_v.at[b]], add=True)

        start_gather(0, 0)

        def pair(j, carry):
            i = j * 2
            start_gather(i + 1, 1)
            drain(i, 0)
            start_gather(i + 2, 0)
            drain(i + 1, 1)
            return carry

        lax.fori_loop(0, nchunks // 2 - 1, pair, 0)
        i = nchunks - 2
        start_gather(i + 1, 1)
        drain(i, 0)
        drain(i + 1, 1)
        plsc.subcore_barrier()
        pltpu.sync_copy(acc_sh.at[pl.ds(r0, _RPT)],
                        out_acc.at[c, pl.ds(r0, _RPT)])

    return agg


@functools.lru_cache(maxsize=None)
def _make_deg(e_pad: int):
    epw = e_pad // _NW
    nchunks = epw // _K
    mesh = plsc.VectorSubcoreMesh(core_axis_name="c", subcore_axis_name="s")

    out_type = jax.ShapeDtypeStruct((_NC, _N_ACC, _D), jnp.float32)
    scratch = (
        pltpu.VMEM((_K,), jnp.int32),
        pltpu.VMEM((_K, _D), jnp.float32),
        pltpu.VMEM_SHARED((_N_ACC, _D), jnp.float32),
    )

    @functools.partial(pl.kernel, out_type=out_type, mesh=mesh,
                       scratch_types=scratch)
    def deg(dsts, zrow, ones, out_deg, dst_v, ones_v, deg_sh):
        c = lax.axis_index("c")
        s = lax.axis_index("s")
        r0 = s * _RPT
        pltpu.sync_copy(zrow.at[pl.ds(r0, _RPT)], deg_sh.at[pl.ds(r0, _RPT)])
        pltpu.sync_copy(ones, ones_v)
        plsc.subcore_barrier()
        base = (c * _NS + s) * epw

        def chunk(i, carry):
            off = pl.multiple_of(base + i * _K, 8)
            pltpu.sync_copy(dsts.at[pl.ds(off, _K)], dst_v)
            pltpu.sync_copy(ones_v, deg_sh.at[dst_v], add=True)
            return carry

        lax.fori_loop(0, nchunks, chunk, 0)
        plsc.subcore_barrier()
        pltpu.sync_copy(deg_sh.at[pl.ds(r0, _RPT)],
                        out_deg.at[c, pl.ds(r0, _RPT)])

    return deg


def _pad_edges(edge_index, e_pad):
    pad = e_pad - edge_index.shape[1]
    fill = jnp.arange(pad, dtype=jnp.int32)
    src = jnp.concatenate([edge_index[0], fill % 512])
    dst = jnp.concatenate([edge_index[1], _N + (fill % _N_DUMP)])
    return src, dst


def _matmul(x, w):
    n, d = x.shape
    f = w.shape[1]
    blk = 1000

    def body(x_ref, w_ref, o_ref):
        o_ref[...] = jnp.dot(x_ref[...], w_ref[...],
                             preferred_element_type=jnp.float32)

    return pl.pallas_call(
        body,
        grid=(n // blk,),
        in_specs=[pl.BlockSpec((blk, d), lambda i: (i, 0)),
                  pl.BlockSpec((d, f), lambda i: (0, 0))],
        out_specs=pl.BlockSpec((blk, f), lambda i: (i, 0)),
        out_shape=jax.ShapeDtypeStruct((n, f), jnp.float32),
    )(x, w)


def _mid(acc, deg, w):
    blk = 1000

    def body(a_ref, d_ref, w_ref, o_ref):
        sm = a_ref[0] + a_ref[1]
        dg = d_ref[0, :, 0:1] + d_ref[1, :, 0:1]
        h = jnp.maximum(sm / jnp.maximum(dg, 1.0), 0.0)
        o_ref[...] = jnp.dot(h, w_ref[...], preferred_element_type=jnp.float32)

    return pl.pallas_call(
        body,
        grid=(_N // blk,),
        in_specs=[pl.BlockSpec((2, blk, _D), lambda i: (0, i, 0)),
                  pl.BlockSpec((2, blk, _D), lambda i: (0, i, 0)),
                  pl.BlockSpec((_D, _D), lambda i: (0, 0))],
        out_specs=pl.BlockSpec((blk, _D), lambda i: (i, 0)),
        out_shape=jax.ShapeDtypeStruct((_N, _D), jnp.float32),
    )(acc, deg, w)


def _final(accl, degl, accp, degp, wq, wk, wv, wo, wfc, wout):
    def body(al, dl, ap, dp, wq_r, wk_r, wv_r, wo_r, wfc_r, wout_r, o_ref):
        hl = jnp.maximum((al[0] + al[1])
                         / jnp.maximum(dl[0, :, 0:1] + dl[1, :, 0:1], 1.0), 0.0)
        hp = jnp.maximum((ap[0] + ap[1])
                         / jnp.maximum(dp[0, :, 0:1] + dp[1, :, 0:1], 1.0), 0.0)
        seq = jnp.concatenate([hl, hp], axis=0)
        scale = _HEAD_DIM ** -0.5
        pooled = jnp.zeros((1, _D), dtype=jnp.float32)
        for h in range(_HEADS):
            q = jnp.dot(seq, wq_r[h], preferred_element_type=jnp.float32)
            k = jnp.dot(seq, wk_r[h], preferred_element_type=jnp.float32)
            v = jnp.dot(seq, wv_r[h], preferred_element_type=jnp.float32)
            sc = lax.dot_general(q, k, (((1,), (1,)), ((), ())),
                                 preferred_element_type=jnp.float32) * scale
            sc = sc - jnp.max(sc, axis=-1, keepdims=True)
            e = jnp.exp(sc)
            p = e / jnp.sum(e, axis=-1, keepdims=True)
            oh = jnp.dot(p, v, preferred_element_type=jnp.float32)
            pooled = pooled + jnp.dot(jnp.mean(oh, axis=0, keepdims=True),
                                      wo_r[h], preferred_element_type=jnp.float32)
        hh = jnp.maximum(jnp.dot(pooled, wfc_r[...],
                                 preferred_element_type=jnp.float32), 0.0)
        logit = jnp.dot(hh, wout_r[...], preferred_element_type=jnp.float32)
        o_ref[...] = 1.0 / (1.0 + jnp.exp(-logit))

    no_idx = lambda i: (0, 0, 0)
    return pl.pallas_call(
        body,
        grid=(1,),
        in_specs=[pl.BlockSpec((2, _HALF, _D), no_idx),
                  pl.BlockSpec((2, _HALF, _D), no_idx),
                  pl.BlockSpec((2, _HALF, _D), no_idx),
                  pl.BlockSpec((2, _HALF, _D), no_idx),
                  pl.BlockSpec((_HEADS, _D, _HEAD_DIM), no_idx),
                  pl.BlockSpec((_HEADS, _D, _HEAD_DIM), no_idx),
                  pl.BlockSpec((_HEADS, _D, _HEAD_DIM), no_idx),
                  pl.BlockSpec((_HEADS, _HEAD_DIM, _D), no_idx),
                  pl.BlockSpec((_D, _HIDDEN), lambda i: (0, 0)),
                  pl.BlockSpec((_HIDDEN, 128), lambda i: (0, 0))],
        out_specs=pl.BlockSpec((1, 128), lambda i: (0, 0)),
        out_shape=jax.ShapeDtypeStruct((1, 128), jnp.float32),
    )(accl, degl, accp, degp, wq, wk, wv, wo, wfc, wout)


def kernel(x_ligand, edge_index_ligand, x_protein, edge_index_protein,
           Wl1, Wl2, Wp1, Wp2, Wqkv, Wo, Wfc, Wout):
    e_l = edge_index_ligand.shape[1]
    e_p = edge_index_protein.shape[1]
    epad_l = _round_up(e_l, 2 * _CHUNK_EDGES)
    epad_p = _round_up(e_p, 2 * _CHUNK_EDGES)

    src_l, dst_l = _pad_edges(edge_index_ligand, epad_l)
    src_p, dst_p = _pad_edges(edge_index_protein, epad_p)

    zrow = jnp.zeros((_N_ACC, _D), jnp.float32)
    ones = jnp.zeros((_K, _D), jnp.float32).at[:, 0].set(1.0)

    deg_l = _make_deg(epad_l)(dst_l, zrow, ones)
    deg_p = _make_deg(epad_p)(dst_p, zrow, ones)

    y_l = _matmul(x_ligand, Wl1)
    y_p = _matmul(x_protein, Wp1)
    acc_l1 = _make_agg(epad_l)(y_l, src_l, dst_l, zrow)
    acc_p1 = _make_agg(epad_p)(y_p, src_p, dst_p, zrow)

    z_l = _mid(acc_l1, deg_l, Wl2)
    z_p = _mid(acc_p1, deg_p, Wp2)

    acc_l2 = _make_agg(epad_l)(z_l, src_l, dst_l, zrow)
    acc_p2 = _make_agg(epad_p)(z_p, src_p, dst_p, zrow)

    wq = Wqkv[:, :_D].reshape(_D, _HEADS, _HEAD_DIM).transpose(1, 0, 2)
    wk = Wqkv[:, _D:2 * _D].reshape(_D, _HEADS, _HEAD_DIM).transpose(1, 0, 2)
    wv = Wqkv[:, 2 * _D:].reshape(_D, _HEADS, _HEAD_DIM).transpose(1, 0, 2)
    wo = Wo.reshape(_HEADS, _HEAD_DIM, _D)
    wout = jnp.pad(Wout, ((0, 0), (0, 127)))

    out = _final(acc_l2, deg_l, acc_p2, deg_p, wq, wk, wv, wo, Wfc, wout)
    return out[0, 0:1]
